# baseline (speedup 1.0000x reference)
import functools

import jax
import jax.numpy as jnp
from jax import lax
from jax.experimental import pallas as pl
from jax.experimental.pallas import tpu as pltpu
from jax.experimental.pallas import tpu_sc as plsc

N = 10000
E = 320000
H = 128
NIN = 256
SCALE = 30.0


BE = 8000


def _edge_mlp_body(x_ref, w1_ref, b1_ref, w2_ref, b2_ref, w3_ref, b3_ref, o_ref):
    x = x_ref[...].astype(jnp.bfloat16)
    m = jnp.dot(x, w1_ref[...], preferred_element_type=jnp.float32) + b1_ref[...]
    m = jnp.maximum(m, 0.0).astype(jnp.bfloat16)
    m = jnp.dot(m, w2_ref[...], preferred_element_type=jnp.float32) + b2_ref[...]
    m = jnp.maximum(m, 0.0).astype(jnp.bfloat16)
    o_ref[...] = jnp.dot(m, w3_ref[...], preferred_element_type=jnp.float32) + b3_ref[...]


def _edge_mlp(h_E, w1t, b1, w2t, b2, w3t, b3, seg, eseg):
    D = H + NIN
    grid = (eseg // BE,)
    blk0 = seg * (eseg // BE)
    return pl.pallas_call(
        _edge_mlp_body,
        grid=grid,
        in_specs=[
            pl.BlockSpec((BE, D), lambda i: (i + blk0, 0)),
            pl.BlockSpec((D, H), lambda i: (0, 0)),
            pl.BlockSpec((1, H), lambda i: (0, 0)),
            pl.BlockSpec((H, H), lambda i: (0, 0)),
            pl.BlockSpec((1, H), lambda i: (0, 0)),
            pl.BlockSpec((H, H), lambda i: (0, 0)),
            pl.BlockSpec((1, H), lambda i: (0, 0)),
        ],
        out_specs=pl.BlockSpec((BE, H), lambda i: (i, 0)),
        out_shape=jax.ShapeDtypeStruct((eseg, H), jnp.float32),
    )(h_E, w1t, b1, w2t, b2, w3t, b3)


NC = 2
NS = 16
NW = NC * NS
CH = 128
NCHUNK = E // CH
CPT = NCHUNK // NW
NREM = NCHUNK - CPT * NW
RTAIL = NW * CPT
NP = 10240
RPT = NP // NS


NSEG = 5
ESEG = E // NSEG
EPW = ESEG // NW
NFULL = EPW // CH
REM = EPW - NFULL * CH


def _make_scatter_body(seg):
    sbase0 = seg * ESEG

    def _scatter_body(m_hbm, src_hbm, out_hbm,
                      acc, idx_a, rows_a, idx_b, rows_b, idx_r, rows_r,
                      sem_ia, sem_ma, sem_ib, sem_mb):
        c = lax.axis_index("c")
        s = lax.axis_index("s")
        wid = c * NS + s
        base = wid * EPW
        r0 = s * RPT

        def issue(i, idx_buf, rows_buf, sem_i, sem_m):
            off = base + i * CH
            pltpu.async_copy(src_hbm.at[pl.ds(sbase0 + off, CH)],
                             idx_buf, sem_i)
            pltpu.async_copy(m_hbm.at[pl.ds(off, CH)], rows_buf, sem_m)

        def wait_and_scatter(idx_buf, rows_buf, sem_i, sem_m):
            pltpu.make_async_copy(src_hbm.at[pl.ds(0, CH)], idx_buf,
                                  sem_i).wait()
            pltpu.make_async_copy(m_hbm.at[pl.ds(0, CH)], rows_buf,
                                  sem_m).wait()
            pltpu.sync_copy(rows_buf, acc.at[idx_buf], add=True)

        issue(0, idx_a, rows_a, sem_ia, sem_ma)

        zv = jnp.zeros((16,), jnp.float32)

        @pl.loop(0, CH)
        def _(r):
            for q in range(H // 16):
                rows_b[r, pl.ds(q * 16, 16)] = zv

        for t in range(RPT // CH):
            pltpu.sync_copy(rows_b, acc.at[pl.ds(r0 + t * CH, CH)])
        plsc.subcore_barrier()

        @pl.loop(0, NFULL - 1, step=2)
        def _(i):
            issue(i + 1, idx_b, rows_b, sem_ib, sem_mb)
            wait_and_scatter(idx_a, rows_a, sem_ia, sem_ma)
            issue(i + 2, idx_a, rows_a, sem_ia, sem_ma)
            wait_and_scatter(idx_b, rows_b, sem_ib, sem_mb)

        wait_and_scatter(idx_a, rows_a, sem_ia, sem_ma)

        off = base + NFULL * CH
        cp_i = pltpu.async_copy(src_hbm.at[pl.ds(sbase0 + off, REM)],
                                idx_r, sem_ib)
        cp_m = pltpu.async_copy(m_hbm.at[pl.ds(off, REM)], rows_r, sem_mb)
        cp_i.wait()
        cp_m.wait()
        pltpu.sync_copy(rows_r, acc.at[idx_r], add=True)

        plsc.subcore_barrier()
        pltpu.sync_copy(acc.at[pl.ds(r0, RPT)], out_hbm.at[c, pl.ds(r0, RPT)])

    return _scatter_body


def _scatter(m_seg, src, seg):
    mesh = plsc.VectorSubcoreMesh(core_axis_name="c", subcore_axis_name="s")
    fn = pl.kernel(
        _make_scatter_body(seg),
        out_type=jax.ShapeDtypeStruct((NC, NP, H), jnp.float32),
        mesh=mesh,
        scratch_types=[
            pltpu.VMEM_SHARED((NP, H), jnp.float32),
            pltpu.VMEM((CH,), jnp.int32),
            pltpu.VMEM((CH, H), jnp.float32),
            pltpu.VMEM((CH,), jnp.int32),
            pltpu.VMEM((CH, H), jnp.float32),
            pltpu.VMEM((REM,), jnp.int32),
            pltpu.VMEM((REM, H), jnp.float32),
            pltpu.SemaphoreType.DMA,
            pltpu.SemaphoreType.DMA,
            pltpu.SemaphoreType.DMA,
            pltpu.SemaphoreType.DMA,
        ],
    )
    return fn(m_seg, src)


BN = 2000
EPS = 1e-5


def _node_body(hv_ref, p0_ref, p1_ref, p2_ref, p3_ref, p4_ref,
               d1_ref, bd1_ref, d2_ref, bd2_ref,
               g1_ref, be1_ref, g2_ref, be2_ref, o_ref):
    f32 = jnp.float32
    dh = jnp.zeros_like(hv_ref[...])
    for p_ref in (p0_ref, p1_ref, p2_ref, p3_ref, p4_ref):
        dh = dh + (p_ref[0].astype(f32) + p_ref[1].astype(f32))
    dh = dh * (1.0 / SCALE)
    h0 = hv_ref[...] + dh
    mu = jnp.mean(h0, axis=1, keepdims=True)
    d = h0 - mu
    var = jnp.mean(d * d, axis=1, keepdims=True)
    h = g1_ref[...] * d * lax.rsqrt(var + EPS) + be1_ref[...]
    t = jnp.dot(h, d1_ref[...], preferred_element_type=jnp.float32) + bd1_ref[...]
    t = jnp.maximum(t, 0.0)
    h2 = h + jnp.dot(t, d2_ref[...], preferred_element_type=jnp.float32) + bd2_ref[...]
    mu2 = jnp.mean(h2, axis=1, keepdims=True)
    d2 = h2 - mu2
    var2 = jnp.mean(d2 * d2, axis=1, keepdims=True)
    o_ref[...] = g2_ref[...] * d2 * lax.rsqrt(var2 + EPS) + be2_ref[...]


def _node_update(h_V, partials, d1t, bd1, d2t, bd2, g1, be1, g2, be2):
    H4 = 4 * H
    grid = (N // BN,)
    return pl.pallas_call(
        _node_body,
        grid=grid,
        in_specs=[
            pl.BlockSpec((BN, H), lambda i: (i, 0)),
        ] + [
            pl.BlockSpec((NC, BN, H), lambda i: (0, i, 0))
            for _ in range(NSEG)
        ] + [
            pl.BlockSpec((H, H4), lambda i: (0, 0)),
            pl.BlockSpec((1, H4), lambda i: (0, 0)),
            pl.BlockSpec((H4, H), lambda i: (0, 0)),
            pl.BlockSpec((1, H), lambda i: (0, 0)),
            pl.BlockSpec((1, H), lambda i: (0, 0)),
            pl.BlockSpec((1, H), lambda i: (0, 0)),
            pl.BlockSpec((1, H), lambda i: (0, 0)),
            pl.BlockSpec((1, H), lambda i: (0, 0)),
        ],
        out_specs=pl.BlockSpec((BN, H), lambda i: (i, 0)),
        out_shape=jax.ShapeDtypeStruct((N, H), jnp.float32),
    )(h_V, *partials, d1t, bd1, d2t, bd2, g1, be1, g2, be2)


def kernel(h_V, h_E, edge_idx, W1, b1, W2, b2, W3, b3, D1, bd1, D2, bd2,
           g1, be1, g2, be2):
    src = edge_idx[0]
    w1t, w2t, w3t = (W1.T.astype(jnp.bfloat16), W2.T.astype(jnp.bfloat16),
                     W3.T.astype(jnp.bfloat16))
    partials = []
    for seg in range(NSEG):
        m_seg = _edge_mlp(h_E, w1t, b1[None, :], w2t, b2[None, :],
                          w3t, b3[None, :], seg, ESEG)
        partials.append(_scatter(m_seg, src, seg))
    return _node_update(h_V, partials,
                        D1.T, bd1[None, :], D2.T, bd2[None, :],
                        g1[None, :], be1[None, :], g2[None, :], be2[None, :])


# speedup vs baseline: 1.0105x; 1.0105x over previous
import functools

import jax
import jax.numpy as jnp
from jax import lax
from jax.experimental import pallas as pl
from jax.experimental.pallas import tpu as pltpu
from jax.experimental.pallas import tpu_sc as plsc

N = 10000
E = 320000
H = 128
NIN = 256
SCALE = 30.0


BE = 6400


def _edge_mlp_body(x_ref, w1_ref, b1_ref, w2_ref, b2_ref, w3_ref, b3_ref, o_ref):
    x = x_ref[...].astype(jnp.bfloat16)
    m = jnp.dot(x, w1_ref[...], preferred_element_type=jnp.float32) + b1_ref[...]
    m = jnp.maximum(m, 0.0).astype(jnp.bfloat16)
    m = jnp.dot(m, w2_ref[...], preferred_element_type=jnp.float32) + b2_ref[...]
    m = jnp.maximum(m, 0.0).astype(jnp.bfloat16)
    o_ref[...] = jnp.dot(m, w3_ref[...], preferred_element_type=jnp.float32) + b3_ref[...]


def _edge_mlp(h_E, w1t, b1, w2t, b2, w3t, b3, seg, eseg):
    D = H + NIN
    grid = (eseg // BE,)
    blk0 = seg * (eseg // BE)
    return pl.pallas_call(
        _edge_mlp_body,
        grid=grid,
        in_specs=[
            pl.BlockSpec((BE, D), lambda i: (i + blk0, 0)),
            pl.BlockSpec((D, H), lambda i: (0, 0)),
            pl.BlockSpec((1, H), lambda i: (0, 0)),
            pl.BlockSpec((H, H), lambda i: (0, 0)),
            pl.BlockSpec((1, H), lambda i: (0, 0)),
            pl.BlockSpec((H, H), lambda i: (0, 0)),
            pl.BlockSpec((1, H), lambda i: (0, 0)),
        ],
        out_specs=pl.BlockSpec((BE, H), lambda i: (i, 0)),
        out_shape=jax.ShapeDtypeStruct((eseg, H), jnp.float32),
    )(h_E, w1t, b1, w2t, b2, w3t, b3)


NC = 2
NS = 16
NW = NC * NS
CH = 128
NCHUNK = E // CH
CPT = NCHUNK // NW
NREM = NCHUNK - CPT * NW
RTAIL = NW * CPT
NP = 10240
RPT = NP // NS


NSEG = 5
ESEG = E // NSEG
EPW = ESEG // NW
NFULL = EPW // CH
REM = EPW - NFULL * CH


def _make_scatter_body(seg):
    sbase0 = seg * ESEG

    def _scatter_body(m_hbm, src_hbm, out_hbm,
                      acc, idx_a, rows_a, idx_b, rows_b, idx_r, rows_r,
                      sem_ia, sem_ma, sem_ib, sem_mb):
        c = lax.axis_index("c")
        s = lax.axis_index("s")
        wid = c * NS + s
        base = wid * EPW
        r0 = s * RPT

        def issue(i, idx_buf, rows_buf, sem_i, sem_m):
            off = base + i * CH
            pltpu.async_copy(src_hbm.at[pl.ds(sbase0 + off, CH)],
                             idx_buf, sem_i)
            pltpu.async_copy(m_hbm.at[pl.ds(off, CH)], rows_buf, sem_m)

        def wait_and_scatter(idx_buf, rows_buf, sem_i, sem_m):
            pltpu.make_async_copy(src_hbm.at[pl.ds(0, CH)], idx_buf,
                                  sem_i).wait()
            pltpu.make_async_copy(m_hbm.at[pl.ds(0, CH)], rows_buf,
                                  sem_m).wait()
            pltpu.sync_copy(rows_buf, acc.at[idx_buf], add=True)

        issue(0, idx_a, rows_a, sem_ia, sem_ma)

        zv = jnp.zeros((16,), jnp.float32)

        @pl.loop(0, CH)
        def _(r):
            for q in range(H // 16):
                rows_b[r, pl.ds(q * 16, 16)] = zv

        for t in range(RPT // CH):
            pltpu.sync_copy(rows_b, acc.at[pl.ds(r0 + t * CH, CH)])
        plsc.subcore_barrier()

        @pl.loop(0, NFULL - 1, step=2)
        def _(i):
            issue(i + 1, idx_b, rows_b, sem_ib, sem_mb)
            wait_and_scatter(idx_a, rows_a, sem_ia, sem_ma)
            issue(i + 2, idx_a, rows_a, sem_ia, sem_ma)
            wait_and_scatter(idx_b, rows_b, sem_ib, sem_mb)

        wait_and_scatter(idx_a, rows_a, sem_ia, sem_ma)

        off = base + NFULL * CH
        cp_i = pltpu.async_copy(src_hbm.at[pl.ds(sbase0 + off, REM)],
                                idx_r, sem_ib)
        cp_m = pltpu.async_copy(m_hbm.at[pl.ds(off, REM)], rows_r, sem_mb)
        cp_i.wait()
        cp_m.wait()
        pltpu.sync_copy(rows_r, acc.at[idx_r], add=True)

        plsc.subcore_barrier()
        pltpu.sync_copy(acc.at[pl.ds(r0, RPT)], out_hbm.at[c, pl.ds(r0, RPT)])

    return _scatter_body


def _scatter(m_seg, src, seg):
    mesh = plsc.VectorSubcoreMesh(core_axis_name="c", subcore_axis_name="s")
    fn = pl.kernel(
        _make_scatter_body(seg),
        out_type=jax.ShapeDtypeStruct((NC, NP, H), jnp.float32),
        mesh=mesh,
        scratch_types=[
            pltpu.VMEM_SHARED((NP, H), jnp.float32),
            pltpu.VMEM((CH,), jnp.int32),
            pltpu.VMEM((CH, H), jnp.float32),
            pltpu.VMEM((CH,), jnp.int32),
            pltpu.VMEM((CH, H), jnp.float32),
            pltpu.VMEM((REM,), jnp.int32),
            pltpu.VMEM((REM, H), jnp.float32),
            pltpu.SemaphoreType.DMA,
            pltpu.SemaphoreType.DMA,
            pltpu.SemaphoreType.DMA,
            pltpu.SemaphoreType.DMA,
        ],
    )
    return fn(m_seg, src)


BN = 2000
EPS = 1e-5


def _node_body(hv_ref, p0_ref, p1_ref, p2_ref, p3_ref, p4_ref,
               d1_ref, bd1_ref, d2_ref, bd2_ref,
               g1_ref, be1_ref, g2_ref, be2_ref, o_ref):
    f32 = jnp.float32
    dh = jnp.zeros_like(hv_ref[...])
    for p_ref in (p0_ref, p1_ref, p2_ref, p3_ref, p4_ref):
        dh = dh + (p_ref[0].astype(f32) + p_ref[1].astype(f32))
    dh = dh * (1.0 / SCALE)
    h0 = hv_ref[...] + dh
    mu = jnp.mean(h0, axis=1, keepdims=True)
    d = h0 - mu
    var = jnp.mean(d * d, axis=1, keepdims=True)
    h = g1_ref[...] * d * lax.rsqrt(var + EPS) + be1_ref[...]
    t = jnp.dot(h, d1_ref[...], preferred_element_type=jnp.float32) + bd1_ref[...]
    t = jnp.maximum(t, 0.0)
    h2 = h + jnp.dot(t, d2_ref[...], preferred_element_type=jnp.float32) + bd2_ref[...]
    mu2 = jnp.mean(h2, axis=1, keepdims=True)
    d2 = h2 - mu2
    var2 = jnp.mean(d2 * d2, axis=1, keepdims=True)
    o_ref[...] = g2_ref[...] * d2 * lax.rsqrt(var2 + EPS) + be2_ref[...]


def _node_update(h_V, partials, d1t, bd1, d2t, bd2, g1, be1, g2, be2):
    H4 = 4 * H
    grid = (N // BN,)
    return pl.pallas_call(
        _node_body,
        grid=grid,
        in_specs=[
            pl.BlockSpec((BN, H), lambda i: (i, 0)),
        ] + [
            pl.BlockSpec((NC, BN, H), lambda i: (0, i, 0))
            for _ in range(NSEG)
        ] + [
            pl.BlockSpec((H, H4), lambda i: (0, 0)),
            pl.BlockSpec((1, H4), lambda i: (0, 0)),
            pl.BlockSpec((H4, H), lambda i: (0, 0)),
            pl.BlockSpec((1, H), lambda i: (0, 0)),
            pl.BlockSpec((1, H), lambda i: (0, 0)),
            pl.BlockSpec((1, H), lambda i: (0, 0)),
            pl.BlockSpec((1, H), lambda i: (0, 0)),
            pl.BlockSpec((1, H), lambda i: (0, 0)),
        ],
        out_specs=pl.BlockSpec((BN, H), lambda i: (i, 0)),
        out_shape=jax.ShapeDtypeStruct((N, H), jnp.float32),
    )(h_V, *partials, d1t, bd1, d2t, bd2, g1, be1, g2, be2)


def kernel(h_V, h_E, edge_idx, W1, b1, W2, b2, W3, b3, D1, bd1, D2, bd2,
           g1, be1, g2, be2):
    src = edge_idx[0]
    w1t, w2t, w3t = (W1.T.astype(jnp.bfloat16), W2.T.astype(jnp.bfloat16),
                     W3.T.astype(jnp.bfloat16))
    partials = []
    for seg in range(NSEG):
        m_seg = _edge_mlp(h_E, w1t, b1[None, :], w2t, b2[None, :],
                          w3t, b3[None, :], seg, ESEG)
        partials.append(_scatter(m_seg, src, seg))
    return _node_update(h_V, partials,
                        D1.T, bd1[None, :], D2.T, bd2[None, :],
                        g1[None, :], be1[None, :], g2[None, :], be2[None, :])
